# Initial kernel scaffold; baseline (speedup 1.0000x reference)
import jax
import jax.numpy as jnp
from jax.experimental import pallas as pl


def kernel(x, edge_index, W1, b1, Wmu, bmu, Wlv, blv):
    raise NotImplementedError("write your pallas kernel here")


# speedup vs baseline: 14.2505x; 14.2505x over previous
import functools

import jax
import jax.numpy as jnp
from jax import lax
from jax.experimental import pallas as pl
from jax.experimental.pallas import tpu as pltpu
from jax.experimental.pallas import tpu_sc as plsc

N = 10000
D = 128
DOUT = 64
NC = 2
NS = 16
NW = NC * NS
CHUNK = 128
ZCH = 128
N_PAD = 10240
DUMMY = N
ZPT = N_PAD // NS

_f32 = jnp.float32


@functools.lru_cache(maxsize=None)
def _deg_pass(nchunk):
    mesh = plsc.VectorSubcoreMesh(
        core_axis_name="c", subcore_axis_name="s",
        num_cores=NC, num_subcores=NS)

    @functools.partial(
        pl.kernel,
        out_type=jax.ShapeDtypeStruct((NC, N_PAD, D), _f32),
        mesh=mesh,
        scratch_types=[
            pltpu.VMEM((nchunk, CHUNK), jnp.int32),
            pltpu.VMEM((CHUNK, D), _f32),
            pltpu.VMEM_SHARED((N_PAD, D), _f32),
        ],
    )
    def deg(didx_hbm, ones_hbm, z_hbm, out_hbm, idx_v, obuf, acc):
        c = lax.axis_index("c")
        s = lax.axis_index("s")
        wid = c * NS + s
        pltpu.sync_copy(ones_hbm, obuf)
        pltpu.sync_copy(didx_hbm.at[wid], idx_v)

        def zero_step(k, carry):
            pltpu.sync_copy(z_hbm, acc.at[pl.ds(s * ZPT + k * ZCH, ZCH)])
            return carry
        lax.fori_loop(0, ZPT // ZCH, zero_step, 0)
        plsc.subcore_barrier()

        def edge_step(j, carry):
            pltpu.sync_copy(obuf, acc.at[idx_v.at[j]], add=True)
            return carry
        lax.fori_loop(0, nchunk, edge_step, 0)
        plsc.subcore_barrier()

        pltpu.sync_copy(acc.at[pl.ds(s * ZPT, ZPT)],
                        out_hbm.at[c, pl.ds(s * ZPT, ZPT)])

    return deg


@functools.lru_cache(maxsize=None)
def _edge_pass(nchunk):
    mesh = plsc.VectorSubcoreMesh(
        core_axis_name="c", subcore_axis_name="s",
        num_cores=NC, num_subcores=NS)

    @functools.partial(
        pl.kernel,
        out_type=jax.ShapeDtypeStruct((NC, N_PAD, D), _f32),
        mesh=mesh,
        scratch_types=[
            pltpu.VMEM((nchunk, CHUNK), jnp.int32),
            pltpu.VMEM((nchunk, CHUNK), jnp.int32),
            pltpu.VMEM((CHUNK, D), _f32),
            pltpu.VMEM_SHARED((N_PAD, D), _f32),
        ],
    )
    def edge(h_hbm, sidx_hbm, didx_hbm, z_hbm, out_hbm,
             sidx_v, didx_v, rows_v, acc):
        c = lax.axis_index("c")
        s = lax.axis_index("s")
        wid = c * NS + s
        pltpu.sync_copy(sidx_hbm.at[wid], sidx_v)
        pltpu.sync_copy(didx_hbm.at[wid], didx_v)

        def zero_step(k, carry):
            pltpu.sync_copy(z_hbm, acc.at[pl.ds(s * ZPT + k * ZCH, ZCH)])
            return carry
        lax.fori_loop(0, ZPT // ZCH, zero_step, 0)
        plsc.subcore_barrier()

        def edge_step(j, carry):
            pltpu.sync_copy(h_hbm.at[sidx_v.at[j]], rows_v)
            pltpu.sync_copy(rows_v, acc.at[didx_v.at[j]], add=True)
            return carry
        lax.fori_loop(0, nchunk, edge_step, 0)
        plsc.subcore_barrier()

        pltpu.sync_copy(acc.at[pl.ds(s * ZPT, ZPT)],
                        out_hbm.at[c, pl.ds(s * ZPT, ZPT)])

    return edge


_BM = 1000
_GRID = (N // _BM,)


def _dinv_of(dp):
    return lax.rsqrt(1.0 + dp[0, :, 0:1] + dp[1, :, 0:1])


def _t1_body(x_ref, w_ref, dp_ref, hs_ref):
    dinv = _dinv_of(dp_ref[...])
    h = jnp.dot(x_ref[...], w_ref[...], preferred_element_type=_f32)
    hs_ref[...] = h * dinv


def _t2_body(p_ref, hs_ref, dp_ref, b1_ref, wml_ref, h2s_ref):
    p = p_ref[...]
    hs = hs_ref[...]
    dinv = _dinv_of(dp_ref[...])
    h1 = jnp.maximum((p[0] + p[1] + hs) * dinv + b1_ref[...], 0.0)
    h2s_ref[...] = jnp.dot(h1, wml_ref[...], preferred_element_type=_f32) * dinv


def _t3_body(p_ref, h2s_ref, dp_ref, bml_ref, o_ref):
    p = p_ref[...]
    dinv = _dinv_of(dp_ref[...])
    o_ref[...] = (p[0] + p[1] + h2s_ref[...]) * dinv + bml_ref[...]


_spec_rows = pl.BlockSpec((_BM, D), lambda i: (i, 0))
_spec_w = pl.BlockSpec((D, D), lambda i: (0, 0))
_spec_dp = pl.BlockSpec((NC, _BM, D), lambda i: (0, i, 0))
_spec_p = pl.BlockSpec((NC, _BM, D), lambda i: (0, i, 0))
_spec_b = pl.BlockSpec((1, D), lambda i: (0, 0))

_t1 = pl.pallas_call(
    _t1_body, grid=_GRID,
    in_specs=[_spec_rows, _spec_w, _spec_dp],
    out_specs=_spec_rows,
    out_shape=jax.ShapeDtypeStruct((N, D), _f32))

_t2 = pl.pallas_call(
    _t2_body, grid=_GRID,
    in_specs=[_spec_p, _spec_rows, _spec_dp, _spec_b, _spec_w],
    out_specs=_spec_rows,
    out_shape=jax.ShapeDtypeStruct((N, D), _f32))

_t3 = pl.pallas_call(
    _t3_body, grid=_GRID,
    in_specs=[_spec_p, _spec_rows, _spec_dp, _spec_b],
    out_specs=_spec_rows,
    out_shape=jax.ShapeDtypeStruct((N, D), _f32))


def kernel(x, edge_index, W1, b1, Wmu, bmu, Wlv, blv):
    src = edge_index[0]
    dst = edge_index[1]
    E = src.shape[0]
    nchunk = -(-E // (NW * CHUNK))
    e_pad = nchunk * NW * CHUNK
    pad = e_pad - E
    src_p = jnp.concatenate(
        [src, jnp.zeros((pad,), jnp.int32)]).reshape(NW, nchunk, CHUNK)
    dst_p = jnp.concatenate(
        [dst, jnp.full((pad,), DUMMY, jnp.int32)]).reshape(NW, nchunk, CHUNK)

    ones128 = jnp.ones((CHUNK, D), _f32)
    z128 = jnp.zeros((ZCH, D), _f32)

    degp = _deg_pass(nchunk)(dst_p, ones128, z128)
    hs = _t1(x, W1, degp)
    part1 = _edge_pass(nchunk)(hs, src_p, dst_p, z128)

    Wml = jnp.concatenate([Wmu, Wlv], axis=1)
    bml = jnp.concatenate([bmu, blv]).reshape(1, D)
    h2s = _t2(part1, hs, degp, b1.reshape(1, D), Wml)
    part2 = _edge_pass(nchunk)(h2s, src_p, dst_p, z128)
    o = _t3(part2, h2s, degp, bml)

    return o[:, :DOUT], o[:, DOUT:]
